# baseline (speedup 1.0000x reference)
import functools

import jax
import jax.numpy as jnp
from jax import lax
from jax.experimental import pallas as pl
from jax.experimental.pallas import tpu as pltpu
from jax.experimental.pallas import tpu_sc as plsc

VOCAB = 1000000
DIM = 64
BATCH = 16384
HIST = 50

NC = 2
NS = 16
NW = NC * NS

BPW = BATCH // NW
B_PER_W = BPW * HIST
CB = 8
R = CB * HIST
STREAMS = ((0, 128), (128, 128), (256, 128), (384, 16))
CHUNKS = BPW // CB
K = 4
T = CHUNKS // K


@functools.partial(
    pl.kernel,
    out_type=jax.ShapeDtypeStruct((BATCH, HIST, DIM), jnp.float32),
    mesh=plsc.VectorSubcoreMesh(core_axis_name="c", subcore_axis_name="s"),
    scratch_types=[
        pltpu.VMEM((B_PER_W,), jnp.int32),
        [pltpu.VMEM((R, DIM), jnp.float32) for _ in range(K)],
        [pltpu.SemaphoreType.DMA for _ in range(K)],
        [pltpu.SemaphoreType.DMA for _ in range(K)],
    ],
    compiler_params=pltpu.CompilerParams(use_tc_tiling_on_sc=False),
)
def _gather_kernel(idx_hbm, table_hbm, out_hbm, idx_v, gbuf, sem_g, sem_o):
    wid = lax.axis_index("s") * NC + lax.axis_index("c")
    base_i = wid * B_PER_W
    base_b = wid * BPW

    def fire_gathers(b, c):
        return [
            pltpu.async_copy(
                table_hbm.at[idx_v.at[pl.ds(c * R + so, sl)]],
                gbuf[b].at[pl.ds(so, sl)],
                sem_g[b],
            )
            for so, sl in STREAMS
        ]

    def fire_outs(b, c):
        for i in range(CB):
            pltpu.async_copy(
                gbuf[b].at[pl.ds(i * HIST, HIST)],
                out_hbm.at[base_b + c * CB + i],
                sem_o[b],
            )

    def drain_outs(b, c):
        for i in range(CB):
            pltpu.make_async_copy(
                gbuf[b].at[pl.ds(i * HIST, HIST)],
                out_hbm.at[base_b + c * CB + i],
                sem_o[b],
            ).wait()

    pltpu.sync_copy(idx_hbm.at[pl.ds(base_i, B_PER_W)], idx_v)

    ds0 = [fire_gathers(b, b) for b in range(K)]
    for b in range(K):
        for d in ds0[b]:
            d.wait()
        fire_outs(b, b)

    def body(t, _):
        c0 = t * K
        ds = []
        for b in range(K):
            drain_outs(b, c0 - K + b)
            ds.append(fire_gathers(b, c0 + b))
        for b in range(K):
            for d in ds[b]:
                d.wait()
            fire_outs(b, c0 + b)
        return 0

    lax.fori_loop(1, T, body, 0)

    for b in range(K):
        drain_outs(b, (T - 1) * K + b)


BLOCKS = VOCAB // 128
BPB = BLOCKS // NW
EXTRA = BLOCKS - BPB * NW
TAIL_R = BLOCKS * 128


@functools.partial(
    pl.kernel,
    out_type=jax.ShapeDtypeStruct((VOCAB // 2, 128), jnp.float32),
    mesh=plsc.VectorSubcoreMesh(core_axis_name="c", subcore_axis_name="s"),
    scratch_types=[
        [pltpu.VMEM((DIM, 128), jnp.float32) for _ in range(2)],
        [pltpu.VMEM((DIM, 128), jnp.float32) for _ in range(2)],
        [pltpu.SemaphoreType.DMA for _ in range(2)],
        [pltpu.SemaphoreType.DMA for _ in range(2)],
    ],
    compiler_params=pltpu.CompilerParams(
        use_tc_tiling_on_sc=True, needs_layout_passes=False
    ),
)
def _fmt_kernel(wt_hbm, stage_hbm, tbuf, sbuf, sem_r, sem_w):
    wid = lax.axis_index("s") * NC + lax.axis_index("c")
    base = wid * BPB
    rows4 = [jax.lax.iota(jnp.int32, 16) + dg * 16 for dg in range(4)]

    def fire_read(b, rb):
        pltpu.async_copy(
            wt_hbm.at[:, pl.ds(rb * 128, 128)], tbuf[b], sem_r[b]
        )

    def wait_read(b, rb):
        pltpu.make_async_copy(
            wt_hbm.at[:, pl.ds(rb * 128, 128)], tbuf[b], sem_r[b]
        ).wait()

    def fire_write(b, rb):
        pltpu.async_copy(
            sbuf[b], stage_hbm.at[pl.ds(rb * 64, DIM)], sem_w[b]
        )

    def drain_write(b, rb):
        pltpu.make_async_copy(
            sbuf[b], stage_hbm.at[pl.ds(rb * 64, DIM)], sem_w[b]
        ).wait()

    def transpose(b, nrows):
        def col(j, _):
            for dg in range(4):
                v = plsc.load_gather(tbuf[b], [rows4[dg], rows4[0] * 0 + j])
                sbuf[b][j >> 1, pl.ds((j & 1) * DIM + dg * 16, 16)] = v
            return 0
        lax.fori_loop(0, nrows, col, 0)

    fire_read(0, base)
    fire_read(1, base + 1)

    def body2(t, _):
        for b in range(2):
            rb = t * 2 + b
            wait_read(b, base + rb)

            @pl.when(rb >= 2)
            def _():
                drain_write(b, base + rb - 2)
            transpose(b, 128)
            fire_write(b, base + rb)

            @pl.when(rb + 2 < BPB)
            def _():
                fire_read(b, base + rb + 2)
        return 0

    lax.fori_loop(0, BPB // 2, body2, 0)
    for b in range(2):
        drain_write(b, base + BPB - 2 + b)

    @pl.when(wid < EXTRA)
    def _():
        rb = NW * BPB + wid
        pltpu.sync_copy(wt_hbm.at[:, pl.ds(rb * 128, 128)], tbuf[0])
        transpose(0, 128)
        pltpu.sync_copy(sbuf[0], stage_hbm.at[pl.ds(rb * 64, DIM)])


def kernel(x, weight):
    idx = x.reshape(BATCH * HIST).astype(jnp.int32)
    stage = _fmt_kernel(weight.T)
    stage = stage.at[TAIL_R // 2:].set(weight[TAIL_R:].reshape(32, 128))
    table = stage.reshape(VOCAB, DIM)
    return _gather_kernel(idx, table)
